# baseline (speedup 1.0000x reference)
import jax
import jax.numpy as jnp
from jax.experimental import pallas as pl
from jax.experimental.pallas import tpu as pltpu

N = 4096
B = 16
BLK = 1024

_CPARAMS = pltpu.CompilerParams(vmem_limit_bytes=int(63 * 2**20),
                               dimension_semantics=("parallel",))


def _rowblock(c):
    return pl.BlockSpec((BLK, c), lambda i: (i, 0))


def _full(shape):
    return pl.BlockSpec(shape, lambda i: tuple(0 for _ in shape))


def _dot(a, b):
    return jnp.dot(a.astype(jnp.bfloat16), b,
                   preferred_element_type=jnp.float32)


def _cast_mm1_body(l_ref, x_ref, lb_ref, o_ref):
    lb = l_ref[...].astype(jnp.bfloat16)
    lb_ref[...] = lb
    o_ref[...] = jnp.dot(lb, x_ref[...].astype(jnp.bfloat16),
                         preferred_element_type=jnp.float32
                         ).astype(jnp.bfloat16)


def _cast_mm1(lap, x):
    c = x.shape[1]
    return pl.pallas_call(
        _cast_mm1_body,
        grid=(N // BLK,),
        in_specs=[_rowblock(N), _full((N, c))],
        out_specs=[_rowblock(N), _rowblock(c)],
        out_shape=[jax.ShapeDtypeStruct((N, N), jnp.bfloat16),
                   jax.ShapeDtypeStruct((N, c), jnp.bfloat16)],
        compiler_params=_CPARAMS,
    )(lap, x)


def _mm1_body(l_ref, x_ref, o_ref):
    o_ref[...] = jnp.dot(l_ref[...], x_ref[...].astype(jnp.bfloat16),
                         preferred_element_type=jnp.float32
                         ).astype(jnp.bfloat16)


def _mm1(lb, x):
    c = x.shape[1]
    return pl.pallas_call(
        _mm1_body,
        grid=(N // BLK,),
        in_specs=[_rowblock(N), _full((N, c))],
        out_specs=_rowblock(c),
        out_shape=jax.ShapeDtypeStruct((N, c), jnp.bfloat16),
        compiler_params=_CPARAMS,
    )(lb, x)


def _epi_body(l_ref, x0_ref, x1_ref, w0_ref, w1_ref, w2_ref, b_ref,
              o_ref, g4_ref, b4_ref):
    i = pl.program_id(0)
    t = jnp.dot(l_ref[...], x1_ref[...], preferred_element_type=jnp.float32)
    x0 = x0_ref[...]
    x2 = 2.0 * t - x0
    x1_blk = x1_ref[pl.ds(i * BLK, BLK), :]
    acc = (_dot(x0, w0_ref[...]) + jnp.dot(x1_blk, w1_ref[...],
                                           preferred_element_type=jnp.float32)
           + _dot(x2, w2_ref[...]) + b_ref[...])
    h = jnp.maximum(acc, 0.0)
    if g4_ref is not None:
        h = _dot(h, g4_ref[...]) + b4_ref[...]
    o_ref[...] = h


def _epi(lb, x0, x1, w0, w1, w2, b, g4=None, b4=None):
    c = x0.shape[1]
    cout = w0.shape[1] if g4 is None else g4.shape[1]
    in_specs = [_rowblock(N), _rowblock(c), _full((N, c)),
                _full(w0.shape), _full(w1.shape), _full(w2.shape),
                _full(b.shape)]
    args = [lb, x0, x1, w0, w1, w2, b]
    if g4 is not None:
        in_specs += [_full(g4.shape), _full(b4.shape)]
        args += [g4, b4]
        def body(l_ref, x0_ref, x1_ref, w0_ref, w1_ref, w2_ref, b_ref,
                 g4_ref, b4_ref, o_ref):
            return _epi_body(l_ref, x0_ref, x1_ref, w0_ref, w1_ref,
                             w2_ref, b_ref, o_ref, g4_ref, b4_ref)
    else:
        def body(l_ref, x0_ref, x1_ref, w0_ref, w1_ref, w2_ref, b_ref,
                 o_ref):
            return _epi_body(l_ref, x0_ref, x1_ref, w0_ref, w1_ref,
                             w2_ref, b_ref, o_ref, None, None)
    return pl.pallas_call(
        body,
        grid=(N // BLK,),
        in_specs=in_specs,
        out_specs=_rowblock(cout),
        out_shape=jax.ShapeDtypeStruct((N, cout), jnp.float32),
        compiler_params=_CPARAMS,
    )(*args)


def _kron_eye(w):
    fin, fout = w.shape
    eye = jnp.eye(B, dtype=w.dtype)
    return jnp.einsum('ab,fo->afbo', eye, w).reshape(B * fin, B * fout)


def kernel(laplacian, inputs, W1, b1, W2, b2, W3, b3, W4, b4):
    x0 = inputs[:, :, 0].T

    w1_0, w1_1, w1_2 = (_kron_eye(W1[k]).astype(jnp.bfloat16)
                        for k in range(3))
    w2_0, w2_1, w2_2 = (_kron_eye(W2[k]).astype(jnp.bfloat16)
                        for k in range(3))
    w3_0, w3_1, w3_2 = (_kron_eye(W3[k]).astype(jnp.bfloat16)
                        for k in range(3))
    g4 = _kron_eye(W4[0]).astype(jnp.bfloat16)
    bb1 = jnp.tile(b1, B)[None, :]
    bb2 = jnp.tile(b2, B)[None, :]
    bb3 = jnp.tile(b3, B)[None, :]
    bb4 = jnp.tile(b4, B)[None, :]

    lb, t1 = _cast_mm1(laplacian, x0)
    y1 = _epi(lb, x0, t1, w1_0, w1_1, w1_2, bb1)

    t1 = _mm1(lb, y1)
    y2 = _epi(lb, y1, t1, w2_0, w2_1, w2_2, bb2)

    t1 = _mm1(lb, y2)
    out = _epi(lb, y2, t1, w3_0, w3_1, w3_2, bb3, g4=g4, b4=bb4)

    return out.T[:, :, None]
